# Initial kernel scaffold; baseline (speedup 1.0000x reference)
import jax
import jax.numpy as jnp
from jax.experimental import pallas as pl


def kernel(x, pos, edge_index, W1, b1, g1, be1, W2, b2, g2, be2, Wl, bl, gl, bel):
    raise NotImplementedError("write your pallas kernel here")


# speedup vs baseline: 2.7482x; 2.7482x over previous
import functools

import jax
import jax.numpy as jnp
from jax import lax
from jax.experimental import pallas as pl
from jax.experimental.pallas import tpu as pltpu
from jax.experimental.pallas import tpu_sc as plsc

N = 10000
D = 128
NEG = -3e38
NEG_TH = -1e38

NPAD = 10240
G = 128
C = 6400
CAP = C + G
L = 16


def _segmax_sc(A, src, dst):
    E = src.shape[0]
    assert E % C == 0
    nchunks = E // C
    info = plsc.get_sparse_core_info()
    NC, NS = info.num_cores, info.num_subcores
    NW = NC * NS
    R = NPAD // NW
    ACCR = R + 8
    TRASH = R

    mesh = plsc.VectorSubcoreMesh(core_axis_name="c", subcore_axis_name="s")

    @functools.partial(
        pl.kernel,
        out_type=jax.ShapeDtypeStruct((NPAD, D), jnp.float32),
        mesh=mesh,
        compiler_params=pltpu.CompilerParams(needs_layout_passes=False),
        scratch_types=[
            pltpu.VMEM((ACCR, D), jnp.float32),
            pltpu.VMEM((C,), jnp.int32),
            pltpu.VMEM((C,), jnp.int32),
            pltpu.VMEM((CAP,), jnp.int32),
            pltpu.VMEM((CAP,), jnp.int32),
            pltpu.VMEM((G,), jnp.int32),
            pltpu.VMEM((G,), jnp.int32),
            pltpu.VMEM((G, D), jnp.float32),
            pltpu.SemaphoreType.DMA,
        ],
    )
    def k(a_hbm, src_hbm, dst_hbm, out_hbm,
          acc, dch, sch, csrc, cdl, gidx, gdl, rows, sem):
        wid = lax.axis_index("s") * NC + lax.axis_index("c")
        lo = wid * R

        negv = jnp.full((L,), NEG, jnp.float32)

        def init_row(i, _):
            for cg in range(D // L):
                acc[i, pl.ds(cg * L, L)] = negv
            return 0

        lax.fori_loop(0, ACCR, init_row, 0)

        def process_group(n0):
            for kk in range(G // L):
                sl = pl.ds(kk * L, L)
                gidx[sl] = csrc[pl.ds(n0 + kk * L, L)]
                gdl[sl] = cdl[pl.ds(n0 + kk * L, L)]
            pltpu.async_copy(a_hbm.at[gidx], rows, sem).wait()

            def acc_grp(t, _):
                dlv = gdl[pl.ds(t * L, L)]
                for ll in range(L):
                    dl = dlv[ll]
                    j = t * L + ll
                    for cg in range(D // L):
                        sl = pl.ds(cg * L, L)
                        acc[dl, sl] = jnp.maximum(acc[dl, sl], rows[j, sl])
                return 0

            lax.fori_loop(0, G // L, acc_grp, 0)

        def chunk_body(c, n):
            base = c * C
            pltpu.sync_copy(dst_hbm.at[pl.ds(base, C)], dch)
            pltpu.sync_copy(src_hbm.at[pl.ds(base, C)], sch)
            nv = jnp.full((L,), n, jnp.int32)

            def filt(i, nv):
                sl = pl.ds(i * L, L)
                d = dch[sl]
                s = sch[sl]
                dl = d - lo
                msk = (dl >= 0) & (dl < R)
                mi = msk.astype(jnp.int32)
                pos = nv + jnp.cumsum(mi) - 1
                plsc.store_scatter(csrc, [pos], s, mask=msk)
                plsc.store_scatter(cdl, [pos], dl, mask=msk)
                return nv + plsc.all_reduce_population_count(msk)

            nv = lax.fori_loop(0, C // L, filt, nv)
            n = jnp.max(nv)

            def drain(n):
                n = n - G
                process_group(n)
                return n

            return lax.while_loop(lambda n: n >= G, drain, n)

        n = lax.fori_loop(0, nchunks, chunk_body, jnp.int32(0))

        lane = lax.iota(jnp.int32, L)
        for kk in range(G // L):
            idxv = kk * L + lane
            mpad = idxv >= n
            plsc.store_scatter(csrc, [idxv], jnp.zeros((L,), jnp.int32),
                               mask=mpad)
            plsc.store_scatter(cdl, [idxv],
                               jnp.full((L,), TRASH, jnp.int32), mask=mpad)
        process_group(jnp.int32(0))

        pltpu.sync_copy(acc.at[pl.ds(0, R)], out_hbm.at[pl.ds(lo, R)])

    return k(A, src, dst)


def _bn(h, gamma, beta):
    mu = jnp.mean(h, axis=0, keepdims=True)
    var = jnp.mean((h - mu) ** 2, axis=0, keepdims=True)
    return (h - mu) / jnp.sqrt(var + 1e-5) * gamma + beta


def _tc_prep_body(x_r, pos2_r, w1x_r, w1p_r, wl_r, bl_r, gl_r, bel_r,
                  a1_r, xlbn_r):
    x = x_r[...]
    pos2 = pos2_r[...]
    a1 = jnp.dot(x, w1x_r[...], preferred_element_type=jnp.float32)
    a1 += jnp.dot(pos2, w1p_r[...], preferred_element_type=jnp.float32)
    a1_r[...] = a1
    xl = jnp.dot(x, wl_r[...], preferred_element_type=jnp.float32) + bl_r[...]
    xlbn_r[...] = _bn(xl, gl_r[...], bel_r[...])


def _tc_mid_body(m1_r, pos2_r, w1p_r, b1_r, g1_r, be1_r, w2x_r, w2p_r, a2_r):
    pos2 = pos2_r[...]
    p1 = jnp.dot(pos2, w1p_r[...], preferred_element_type=jnp.float32)
    m1 = m1_r[...]
    h = jnp.where(m1 <= NEG_TH, 0.0, m1 - p1 + b1_r[...])
    h = jnp.maximum(_bn(h, g1_r[...], be1_r[...]), 0.0)
    a2 = jnp.dot(h, w2x_r[...], preferred_element_type=jnp.float32)
    a2 += jnp.dot(pos2, w2p_r[...], preferred_element_type=jnp.float32)
    a2_r[...] = a2


def _tc_final_body(m2_r, pos2_r, w2p_r, b2_r, g2_r, be2_r, xlbn_r, out_r):
    p2 = jnp.dot(pos2_r[...], w2p_r[...], preferred_element_type=jnp.float32)
    m2 = m2_r[...]
    h = jnp.where(m2 <= NEG_TH, 0.0, m2 - p2 + b2_r[...])
    h = _bn(h, g2_r[...], be2_r[...])
    out_r[...] = jnp.maximum(h + xlbn_r[...], 0.0)


def _f32_out(*shapes):
    return [jax.ShapeDtypeStruct(s, jnp.float32) for s in shapes]


def kernel(x, pos, edge_index, W1, b1, g1, be1, W2, b2, g2, be2,
           Wl, bl, gl, bel):
    pos2 = pos[:, :2]
    src = edge_index[0]
    dst = edge_index[1]
    W1x, W1p = W1[:D], W1[D:]
    W2x, W2p = W2[:D], W2[D:]
    b1r, g1r, be1r = b1.reshape(1, D), g1.reshape(1, D), be1.reshape(1, D)
    b2r, g2r, be2r = b2.reshape(1, D), g2.reshape(1, D), be2.reshape(1, D)
    blr, glr, belr = bl.reshape(1, D), gl.reshape(1, D), bel.reshape(1, D)

    a1, xlbn = pl.pallas_call(
        _tc_prep_body,
        out_shape=_f32_out((N, D), (N, D)),
    )(x, pos2, W1x, W1p, Wl, blr, glr, belr)

    m1 = _segmax_sc(a1, src, dst)[:N]

    a2, = pl.pallas_call(
        _tc_mid_body,
        out_shape=_f32_out((N, D)),
    )(m1, pos2, W1p, b1r, g1r, be1r, W2x, W2p)

    m2 = _segmax_sc(a2, src, dst)[:N]

    out, = pl.pallas_call(
        _tc_final_body,
        out_shape=_f32_out((N, D)),
    )(m2, pos2, W2p, b2r, g2r, be2r, xlbn)

    return out
